# baseline (speedup 1.0000x reference)
import dataclasses
import functools

import jax
import jax.numpy as jnp
from jax import lax
from jax.experimental import pallas as pl
from jax.experimental.pallas import tpu as pltpu
from jax.experimental.pallas import tpu_sc as plsc

_L = 16
_NC, _NS = 2, 16
_NW = _NC * _NS
_W = 32
_RING = 2


def _lse_body(tab_ref, lse_ref):
    x = tab_ref[...]
    m = jnp.max(x, axis=1, keepdims=True)
    s = jnp.sum(jnp.exp(x - m), axis=1, keepdims=True)
    lse_ref[...] = jnp.log(s) + m


def _loss_body(n, part_ref, loss_ref):
    loss_ref[...] = (jnp.sum(part_ref[...]) / jnp.float32(n)).reshape(1, 1)


def _unpad_first_body(chunk_ref, out_ref):
    out_ref[...] = chunk_ref[:, : out_ref.shape[1]]


def _unpad_next_body(prev_ref, chunk_ref, out_ref):
    del prev_ref
    out_ref[...] = chunk_ref[:, : out_ref.shape[1]]


@functools.lru_cache(maxsize=None)
def _make_sc_main(n, v):
    mesh = plsc.VectorSubcoreMesh(core_axis_name="c", subcore_axis_name="s")
    cp = pltpu.CompilerParams()
    for _f, _v in (("needs_layout_passes", False),
                   ("use_tc_tiling_on_sc", True)):
        if _f in pltpu.CompilerParams.__dataclass_fields__:
            cp = dataclasses.replace(cp, **{_f: _v})

    ni = n // _NW
    steps = ni // _W
    assert steps % _RING == 0

    @functools.partial(
        pl.kernel,
        compiler_params=cp,
        out_type=[
            jax.ShapeDtypeStruct((n, 1024), jnp.float32),
            jax.ShapeDtypeStruct((_NW, 128), jnp.float32),
        ],
        mesh=mesh,
        scratch_types=[
            pltpu.VMEM((ni,), jnp.int32),
            pltpu.VMEM((ni,), jnp.int32),
            pltpu.VMEM((1024,), jnp.float32),
            pltpu.VMEM((128,), jnp.float32),
            pltpu.VMEM((_W, 1024), jnp.float32),
            pltpu.VMEM((_W, 1024), jnp.float32),
            pltpu.SemaphoreType.DMA,
            pltpu.SemaphoreType.DMA,
        ],
    )
    def sc_main(table_hbm, idx_hbm, tgt_hbm, lse_hbm, out_hbm, part_hbm,
                idx_v, tgt_v, lse_v, acc_v, buf0, buf1, sem0, sem1):
        wid = lax.axis_index("s") * _NC + lax.axis_index("c")
        base = wid * ni
        pltpu.sync_copy(idx_hbm.at[pl.ds(base, ni)], idx_v)
        pltpu.sync_copy(tgt_hbm.at[pl.ds(base, ni)], tgt_v)
        pltpu.sync_copy(lse_hbm, lse_v)

        @pl.loop(0, 128, step=_L)
        def _(j):
            acc_v[pl.ds(j, _L)] = jnp.zeros((_L,), jnp.float32)

        bufs = (buf0, buf1)
        sems = (sem0, sem1)

        @pl.loop(0, steps // _RING)
        def _(g):
            for b in range(_RING):
                step = g * _RING + b
                buf, sem = bufs[b], sems[b]

                @pl.when(g > 0)
                def _():
                    pltpu.make_async_copy(
                        buf,
                        out_hbm.at[pl.ds(base + (step - _RING) * _W, _W)],
                        sem,
                    ).wait()

                pltpu.sync_copy(
                    table_hbm.at[idx_v.at[pl.ds(step * _W, _W)]], buf)

                @pl.loop(0, _W, step=_L)
                def _(j):
                    rvec = j + lax.iota(jnp.int32, _L)
                    ivec = idx_v[pl.ds(step * _W + j, _L)]
                    tvec = tgt_v[pl.ds(step * _W + j, _L)]
                    lse_vals = plsc.load_gather(lse_v, [ivec])
                    elems = plsc.load_gather(buf, [rvec, tvec])
                    acc_v[pl.ds(0, _L)] = (
                        acc_v[pl.ds(0, _L)] + lse_vals - elems)

                pltpu.async_copy(
                    buf, out_hbm.at[pl.ds(base + step * _W, _W)], sem)

        for b in range(_RING):
            pltpu.make_async_copy(
                bufs[b],
                out_hbm.at[pl.ds(base + (steps - _RING + b) * _W, _W)],
                sems[b],
            ).wait()

        pltpu.sync_copy(acc_v, part_hbm.at[wid])

    return sc_main


_K = 4
_R = 512


def kernel(idx, targets, table):
    b, t = idx.shape
    v = table.shape[0]
    n = b * t
    nk = n // _K
    idx_f = idx.reshape(n).astype(jnp.int32)
    tgt_f = targets.reshape(n).astype(jnp.int32)

    lse = pl.pallas_call(
        _lse_body,
        out_shape=jax.ShapeDtypeStruct((v, 1), jnp.float32),
    )(table)
    lse_p = jnp.pad(lse.reshape(v), (0, 1024 - v))
    table_p = jnp.pad(table, ((0, 0), (0, 1024 - v)))

    sc_chunk = _make_sc_main(nk, v)
    chunks, parts = [], []
    for k in range(_K):
        ck, pk = sc_chunk(table_p, idx_f[k * nk:(k + 1) * nk],
                          tgt_f[k * nk:(k + 1) * nk], lse_p)
        chunks.append(ck)
        parts.append(pk)

    logits = pl.pallas_call(
        _unpad_first_body,
        grid=(nk // _R,),
        in_specs=[pl.BlockSpec((_R, 1024), lambda i: (i, 0))],
        out_specs=pl.BlockSpec((_R, v), lambda i: (i, 0)),
        out_shape=jax.ShapeDtypeStruct((n, v), jnp.float32),
    )(chunks[0])
    for k in range(1, _K):
        logits = pl.pallas_call(
            _unpad_next_body,
            grid=(nk // _R,),
            in_specs=[pl.BlockSpec(memory_space=pl.ANY),
                      pl.BlockSpec((_R, 1024), lambda i: (i, 0))],
            out_specs=pl.BlockSpec(
                (_R, v), lambda i, _o=k * (nk // _R): (i + _o, 0)),
            out_shape=jax.ShapeDtypeStruct((n, v), jnp.float32),
            input_output_aliases={0: 0},
        )(logits, chunks[k])

    loss = pl.pallas_call(
        functools.partial(_loss_body, n),
        out_shape=jax.ShapeDtypeStruct((1, 1), jnp.float32),
    )(jnp.stack(parts))

    return loss[0, 0], logits


# speedup vs baseline: 1.1254x; 1.1254x over previous
import dataclasses
import functools

import jax
import jax.numpy as jnp
from jax import lax
from jax.experimental import pallas as pl
from jax.experimental.pallas import tpu as pltpu
from jax.experimental.pallas import tpu_sc as plsc

_L = 16
_NC, _NS = 2, 16
_NW = _NC * _NS
_W = 32
_RING = 2


def _lse_body(tab_ref, lse_ref):
    x = tab_ref[...]
    m = jnp.max(x, axis=1, keepdims=True)
    s = jnp.sum(jnp.exp(x - m), axis=1, keepdims=True)
    lse_ref[...] = jnp.log(s) + m


def _loss_body(n, part_ref, loss_ref):
    loss_ref[...] = (jnp.sum(part_ref[...]) / jnp.float32(n)).reshape(1, 1)


def _unpad_first_body(chunk_ref, out_ref):
    out_ref[...] = chunk_ref[:, : out_ref.shape[1]]


def _unpad_next_body(prev_ref, chunk_ref, out_ref):
    del prev_ref
    out_ref[...] = chunk_ref[:, : out_ref.shape[1]]


@functools.lru_cache(maxsize=None)
def _make_sc_main(n, v):
    mesh = plsc.VectorSubcoreMesh(core_axis_name="c", subcore_axis_name="s")
    cp = pltpu.CompilerParams()
    for _f, _v in (("needs_layout_passes", False),
                   ("use_tc_tiling_on_sc", True)):
        if _f in pltpu.CompilerParams.__dataclass_fields__:
            cp = dataclasses.replace(cp, **{_f: _v})

    ni = n // _NW
    steps = ni // _W
    assert steps % _RING == 0

    vmain = (v // 128) * 128
    vtail = v - vmain

    @functools.partial(
        pl.kernel,
        compiler_params=cp,
        out_type=[
            jax.ShapeDtypeStruct((n, v), jnp.float32),
            jax.ShapeDtypeStruct((_NW, 128), jnp.float32),
        ],
        mesh=mesh,
        scratch_types=[
            pltpu.VMEM((ni,), jnp.int32),
            pltpu.VMEM((ni,), jnp.int32),
            pltpu.VMEM((1024,), jnp.float32),
            pltpu.VMEM((128,), jnp.float32),
            pltpu.VMEM((_W, v), jnp.float32),
            pltpu.VMEM((_W, v), jnp.float32),
            pltpu.VMEM((_W, 128), jnp.float32),
            pltpu.VMEM((_W, 128), jnp.float32),
            pltpu.SemaphoreType.DMA,
            pltpu.SemaphoreType.DMA,
        ],
    )
    def sc_main(tmain_hbm, ttail_hbm, idx_hbm, tgt_hbm, lse_hbm, out_hbm,
                part_hbm, idx_v, tgt_v, lse_v, acc_v, buf0, buf1, tbuf0,
                tbuf1, sem0, sem1):
        wid = lax.axis_index("s") * _NC + lax.axis_index("c")
        base = wid * ni
        pltpu.sync_copy(idx_hbm.at[pl.ds(base, ni)], idx_v)
        pltpu.sync_copy(tgt_hbm.at[pl.ds(base, ni)], tgt_v)
        pltpu.sync_copy(lse_hbm, lse_v)

        @pl.loop(0, 128, step=_L)
        def _(j):
            acc_v[pl.ds(j, _L)] = jnp.zeros((_L,), jnp.float32)

        bufs = (buf0, buf1)
        sems = (sem0, sem1)

        tbufs = (tbuf0, tbuf1)

        @pl.loop(0, steps // _RING)
        def _(g):
            for b in range(_RING):
                step = g * _RING + b
                buf, tbuf, sem = bufs[b], tbufs[b], sems[b]

                @pl.when(g > 0)
                def _():
                    pltpu.make_async_copy(
                        buf,
                        out_hbm.at[pl.ds(base + (step - _RING) * _W, _W)],
                        sem,
                    ).wait()

                idx_ref = idx_v.at[pl.ds(step * _W, _W)]
                pltpu.sync_copy(tmain_hbm.at[idx_ref],
                                buf.at[:, pl.ds(0, vmain)])
                pltpu.sync_copy(ttail_hbm.at[idx_ref], tbuf)

                @pl.loop(0, _W)
                def _(r):
                    for j in range(vtail // _L):
                        buf[r, pl.ds(vmain + j * _L, _L)] = (
                            tbuf[r, pl.ds(j * _L, _L)])
                    if vtail % _L:
                        buf[r, pl.ds(v - _L, _L)] = (
                            tbuf[r, pl.ds(vtail - _L, _L)])

                @pl.loop(0, _W, step=_L)
                def _(j):
                    rvec = j + lax.iota(jnp.int32, _L)
                    ivec = idx_v[pl.ds(step * _W + j, _L)]
                    tvec = tgt_v[pl.ds(step * _W + j, _L)]
                    lse_vals = plsc.load_gather(lse_v, [ivec])
                    elems = plsc.load_gather(buf, [rvec, tvec])
                    acc_v[pl.ds(0, _L)] = (
                        acc_v[pl.ds(0, _L)] + lse_vals - elems)

                pltpu.async_copy(
                    buf, out_hbm.at[pl.ds(base + step * _W, _W)], sem)

        for b in range(_RING):
            pltpu.make_async_copy(
                bufs[b],
                out_hbm.at[pl.ds(base + (steps - _RING + b) * _W, _W)],
                sems[b],
            ).wait()

        pltpu.sync_copy(acc_v, part_hbm.at[wid])

    return sc_main


_K = 4
_R = 512


def kernel(idx, targets, table):
    b, t = idx.shape
    v = table.shape[0]
    n = b * t
    nk = n // _K
    idx_f = idx.reshape(n).astype(jnp.int32)
    tgt_f = targets.reshape(n).astype(jnp.int32)

    lse = pl.pallas_call(
        _lse_body,
        out_shape=jax.ShapeDtypeStruct((v, 1), jnp.float32),
    )(table)
    lse_p = jnp.pad(lse.reshape(v), (0, 1024 - v))
    vmain = (v // 128) * 128
    tmain = table[:, :vmain]
    ttail = jnp.pad(table[:, vmain:], ((0, 0), (0, 128 - (v - vmain))))

    logits, partials = _make_sc_main(n, v)(tmain, ttail, idx_f, tgt_f, lse_p)

    loss = pl.pallas_call(
        functools.partial(_loss_body, n),
        out_shape=jax.ShapeDtypeStruct((1, 1), jnp.float32),
    )(partials)

    return loss[0, 0], logits


# speedup vs baseline: 1.5521x; 1.3792x over previous
import dataclasses
import functools

import jax
import jax.numpy as jnp
from jax import lax
from jax.experimental import pallas as pl
from jax.experimental.pallas import tpu as pltpu
from jax.experimental.pallas import tpu_sc as plsc

_L = 16
_NC, _NS = 2, 16
_NW = _NC * _NS
_W = 32
_RING = 2


def _lse_body(tab_ref, lse_ref):
    x = tab_ref[...]
    m = jnp.max(x, axis=1, keepdims=True)
    s = jnp.sum(jnp.exp(x - m), axis=1, keepdims=True)
    lse_ref[...] = jnp.log(s) + m


def _loss_body(n, part_ref, loss_ref):
    loss_ref[...] = (jnp.sum(part_ref[...]) / jnp.float32(n)).reshape(1, 1)


def _unpad_first_body(chunk_ref, out_ref):
    out_ref[...] = chunk_ref[:, : out_ref.shape[1]]


def _unpad_next_body(prev_ref, chunk_ref, out_ref):
    del prev_ref
    out_ref[...] = chunk_ref[:, : out_ref.shape[1]]


@functools.lru_cache(maxsize=None)
def _make_sc_main(n, v):
    mesh = plsc.VectorSubcoreMesh(core_axis_name="c", subcore_axis_name="s")
    cp = pltpu.CompilerParams()
    for _f, _v in (("needs_layout_passes", False),
                   ("use_tc_tiling_on_sc", True)):
        if _f in pltpu.CompilerParams.__dataclass_fields__:
            cp = dataclasses.replace(cp, **{_f: _v})

    ni = n // _NW
    steps = ni // _W
    assert steps % _RING == 0

    @functools.partial(
        pl.kernel,
        compiler_params=cp,
        out_type=[
            jax.ShapeDtypeStruct((n, 1024), jnp.float32),
            jax.ShapeDtypeStruct((_NW, 128), jnp.float32),
        ],
        mesh=mesh,
        scratch_types=[
            pltpu.VMEM((ni,), jnp.int32),
            pltpu.VMEM((ni,), jnp.int32),
            pltpu.VMEM((1024,), jnp.float32),
            pltpu.VMEM((128,), jnp.float32),
            pltpu.VMEM((_W, 1024), jnp.float32),
            pltpu.VMEM((_W, 1024), jnp.float32),
            pltpu.SemaphoreType.DMA,
            pltpu.SemaphoreType.DMA,
            pltpu.SemaphoreType.DMA,
            pltpu.SemaphoreType.DMA,
        ],
    )
    def sc_main(table_hbm, idx_hbm, tgt_hbm, lse_hbm, out_hbm,
                part_hbm, idx_v, tgt_v, lse_v, acc_v, buf0, buf1,
                sem0, sem1, gsem0, gsem1):
        wid = lax.axis_index("s") * _NC + lax.axis_index("c")
        base = wid * ni
        pltpu.sync_copy(idx_hbm.at[pl.ds(base, ni)], idx_v)
        pltpu.sync_copy(tgt_hbm.at[pl.ds(base, ni)], tgt_v)
        pltpu.sync_copy(lse_hbm, lse_v)

        @pl.loop(0, 128, step=_L)
        def _(j):
            acc_v[pl.ds(j, _L)] = jnp.zeros((_L,), jnp.float32)

        bufs = (buf0, buf1)
        wsems = (sem0, sem1)
        gsems = (gsem0, gsem1)

        pltpu.async_copy(
            table_hbm.at[idx_v.at[pl.ds(0, _W)]], buf0, gsem0)

        @pl.loop(0, steps // _RING)
        def _(g):
            for b in range(_RING):
                step = g * _RING + b
                buf, wsem, gsem = bufs[b], wsems[b], gsems[b]
                b1 = 1 - b
                nbuf, nwsem, ngsem = bufs[b1], wsems[b1], gsems[b1]

                def _advance(step=step, nbuf=nbuf, nwsem=nwsem,
                             ngsem=ngsem, drain=True):
                    if drain:
                        pltpu.make_async_copy(
                            nbuf,
                            out_hbm.at[pl.ds(base + (step - 1) * _W, _W)],
                            nwsem,
                        ).wait()
                    pltpu.async_copy(
                        table_hbm.at[idx_v.at[pl.ds((step + 1) * _W, _W)]],
                        nbuf, ngsem)

                if b == 0:
                    @pl.when(g > 0)
                    def _():
                        _advance()

                    @pl.when(g == 0)
                    def _():
                        _advance(drain=False)
                else:
                    @pl.when(g < steps // _RING - 1)
                    def _():
                        _advance()

                pltpu.make_async_copy(
                    table_hbm.at[idx_v.at[pl.ds(step * _W, _W)]], buf, gsem
                ).wait()

                @pl.loop(0, _W, step=_L)
                def _(j):
                    rvec = j + lax.iota(jnp.int32, _L)
                    ivec = idx_v[pl.ds(step * _W + j, _L)]
                    tvec = tgt_v[pl.ds(step * _W + j, _L)]
                    lse_vals = plsc.load_gather(lse_v, [ivec])
                    elems = plsc.load_gather(buf, [rvec, tvec])
                    acc_v[pl.ds(0, _L)] = (
                        acc_v[pl.ds(0, _L)] + lse_vals - elems)

                pltpu.async_copy(
                    buf, out_hbm.at[pl.ds(base + step * _W, _W)], wsem)

        for b in range(_RING):
            pltpu.make_async_copy(
                bufs[b],
                out_hbm.at[pl.ds(base + (steps - _RING + b) * _W, _W)],
                wsems[b],
            ).wait()

        pltpu.sync_copy(acc_v, part_hbm.at[wid])

    return sc_main


_K = 4
_R = 512


def kernel(idx, targets, table):
    b, t = idx.shape
    v = table.shape[0]
    n = b * t
    nk = n // _K
    idx_f = idx.reshape(n).astype(jnp.int32)
    tgt_f = targets.reshape(n).astype(jnp.int32)

    lse = pl.pallas_call(
        _lse_body,
        out_shape=jax.ShapeDtypeStruct((v, 1), jnp.float32),
    )(table)
    lse_p = jnp.pad(lse.reshape(v), (0, 1024 - v))
    table_p = jnp.pad(table, ((0, 0), (0, 1024 - v)))

    logits_p, partials = _make_sc_main(n, v)(table_p, idx_f, tgt_f, lse_p)
    logits = logits_p[:, :v]

    loss = pl.pallas_call(
        functools.partial(_loss_body, n),
        out_shape=jax.ShapeDtypeStruct((1, 1), jnp.float32),
    )(partials)

    return loss[0, 0], logits
